# baseline (speedup 1.0000x reference)
import math

import jax
import jax.numpy as jnp
from jax import lax
from jax.experimental import pallas as pl
from jax.experimental.pallas import tpu as pltpu
from jax.experimental.pallas import tpu_sc as plsc

VOCAB = 1000000
EMBED = 64
WINDOW = 200
BATCH = 4096

NC, NS, LANES = 2, 16, 16
NW = NC * NS
SEQ_PER_W = BATCH // NW
ROWS = WINDOW
G0 = 128
G1 = ROWS - G0
VECS_PER_ROW = EMBED // LANES
SCALE = math.sqrt(EMBED)


def _body(x_hbm, table_hbm, pe_hbm, out_hbm,
          idx_v, pe_v, rows0, rows1, out0, out1,
          gsem0, gsem1, wsem0, wsem1):
    wid = lax.axis_index("s") * NC + lax.axis_index("c")
    seq0 = wid * SEQ_PER_W

    pltpu.sync_copy(pe_hbm, pe_v)
    pltpu.sync_copy(x_hbm.at[pl.ds(seq0 * WINDOW, SEQ_PER_W * WINDOW)], idx_v)

    bufs = ((rows0, out0, gsem0, wsem0), (rows1, out1, gsem1, wsem1))

    def start_gather(c, rows_b, gsem_b):
        off = c * ROWS
        pltpu.async_copy(table_hbm.at[idx_v.at[pl.ds(off, G0)]],
                         rows_b.at[pl.ds(0, G0)], gsem_b)
        pltpu.async_copy(table_hbm.at[idx_v.at[pl.ds(off + G0, G1)]],
                         rows_b.at[pl.ds(G0, G1)], gsem_b)

    def wait_gather(c, rows_b, gsem_b):
        off = c * ROWS
        pltpu.make_async_copy(table_hbm.at[idx_v.at[pl.ds(off, G0)]],
                              rows_b.at[pl.ds(0, G0)], gsem_b).wait()
        pltpu.make_async_copy(table_hbm.at[idx_v.at[pl.ds(off + G0, G1)]],
                              rows_b.at[pl.ds(G0, G1)], gsem_b).wait()

    for b, (rows_b, out_b, gsem_b, wsem_b) in enumerate(bufs):
        start_gather(b, rows_b, gsem_b)

    def step(j, _):
        for b, (rows_b, out_b, gsem_b, wsem_b) in enumerate(bufs):
            c = 2 * j + b
            wait_gather(c, rows_b, gsem_b)

            @pl.when(j >= 1)
            def _():
                pltpu.make_async_copy(
                    out_b, out_hbm.at[seq0 + c - 2], wsem_b).wait()

            def row_step(r, _):
                for k in range(VECS_PER_ROW):
                    sl = pl.ds(k * LANES, LANES)
                    out_b[r, sl] = (rows_b[r, sl] * SCALE
                                    + pe_v[r, sl])
                return ()

            lax.fori_loop(0, ROWS, row_step, (), unroll=4)

            @pl.when(c + 2 < SEQ_PER_W)
            def _():
                start_gather(c + 2, rows_b, gsem_b)

            pltpu.async_copy(out_b, out_hbm.at[seq0 + c], wsem_b)
        return ()

    lax.fori_loop(0, SEQ_PER_W // 2, step, (), unroll=False)

    for b, (rows_b, out_b, gsem_b, wsem_b) in enumerate(bufs):
        c = SEQ_PER_W - 2 + b
        pltpu.make_async_copy(out_b, out_hbm.at[seq0 + c], wsem_b).wait()


def kernel(x, table, pos_encoding):
    xf = x.reshape(BATCH * WINDOW)
    pe = pos_encoding[:WINDOW, :]

    mesh = plsc.VectorSubcoreMesh(
        core_axis_name="c", subcore_axis_name="s",
        num_cores=NC, num_subcores=NS)

    out = pl.kernel(
        _body,
        out_type=jax.ShapeDtypeStruct((BATCH, WINDOW, EMBED), jnp.float32),
        mesh=mesh,
        compiler_params=pltpu.CompilerParams(use_tc_tiling_on_sc=False),
        scratch_types=[
            pltpu.VMEM((SEQ_PER_W * WINDOW,), jnp.int32),
            pltpu.VMEM((WINDOW, EMBED), jnp.float32),
            pltpu.VMEM((ROWS, EMBED), jnp.float32),
            pltpu.VMEM((ROWS, EMBED), jnp.float32),
            pltpu.VMEM((ROWS, EMBED), jnp.float32),
            pltpu.VMEM((ROWS, EMBED), jnp.float32),
            pltpu.SemaphoreType.DMA,
            pltpu.SemaphoreType.DMA,
            pltpu.SemaphoreType.DMA,
            pltpu.SemaphoreType.DMA,
        ],
    )(xf, table, pe)
    return out


# speedup vs baseline: 1.0011x; 1.0011x over previous
import math

import jax
import jax.numpy as jnp
from jax import lax
from jax.experimental import pallas as pl
from jax.experimental.pallas import tpu as pltpu
from jax.experimental.pallas import tpu_sc as plsc

VOCAB = 1000000
EMBED = 64
WINDOW = 200
BATCH = 4096

NC, NS, LANES = 2, 16, 16
NW = NC * NS
SEQ_PER_W = BATCH // NW
ROWS = WINDOW
G0 = 128
G1 = ROWS - G0
VECS_PER_ROW = EMBED // LANES
SCALE = math.sqrt(EMBED)


def _body(x_hbm, table_hbm, pe_hbm, out_hbm,
          idx_v, pe_v, rows0, rows1, out0, out1,
          gsem0, gsem1, wsem0, wsem1):
    wid = lax.axis_index("s") * NC + lax.axis_index("c")
    seq0 = wid * SEQ_PER_W

    pltpu.sync_copy(pe_hbm, pe_v)
    pltpu.sync_copy(x_hbm.at[pl.ds(seq0 * WINDOW, SEQ_PER_W * WINDOW)], idx_v)

    bufs = ((rows0, out0, gsem0, wsem0), (rows1, out1, gsem1, wsem1))

    def start_gather(c, rows_b, gsem_b):
        off = c * ROWS
        pltpu.async_copy(table_hbm.at[idx_v.at[pl.ds(off, ROWS)]],
                         rows_b, gsem_b)

    def wait_gather(c, rows_b, gsem_b):
        off = c * ROWS
        pltpu.make_async_copy(table_hbm.at[idx_v.at[pl.ds(off, ROWS)]],
                              rows_b, gsem_b).wait()

    for b, (rows_b, out_b, gsem_b, wsem_b) in enumerate(bufs):
        start_gather(b, rows_b, gsem_b)

    def step(j, _):
        for b, (rows_b, out_b, gsem_b, wsem_b) in enumerate(bufs):
            c = 2 * j + b
            wait_gather(c, rows_b, gsem_b)

            @pl.when(j >= 1)
            def _():
                pltpu.make_async_copy(
                    out_b, out_hbm.at[seq0 + c - 2], wsem_b).wait()

            def row_step(r, _):
                for k in range(VECS_PER_ROW):
                    sl = pl.ds(k * LANES, LANES)
                    out_b[r, sl] = (rows_b[r, sl] * SCALE
                                    + pe_v[r, sl])
                return ()

            lax.fori_loop(0, ROWS, row_step, (), unroll=4)

            @pl.when(c + 2 < SEQ_PER_W)
            def _():
                start_gather(c + 2, rows_b, gsem_b)

            pltpu.async_copy(out_b, out_hbm.at[seq0 + c], wsem_b)
        return ()

    lax.fori_loop(0, SEQ_PER_W // 2, step, (), unroll=False)

    for b, (rows_b, out_b, gsem_b, wsem_b) in enumerate(bufs):
        c = SEQ_PER_W - 2 + b
        pltpu.make_async_copy(out_b, out_hbm.at[seq0 + c], wsem_b).wait()


def kernel(x, table, pos_encoding):
    xf = x.reshape(BATCH * WINDOW)
    pe = pos_encoding[:WINDOW, :]

    mesh = plsc.VectorSubcoreMesh(
        core_axis_name="c", subcore_axis_name="s",
        num_cores=NC, num_subcores=NS)

    out = pl.kernel(
        _body,
        out_type=jax.ShapeDtypeStruct((BATCH, WINDOW, EMBED), jnp.float32),
        mesh=mesh,
        compiler_params=pltpu.CompilerParams(use_tc_tiling_on_sc=False),
        scratch_types=[
            pltpu.VMEM((SEQ_PER_W * WINDOW,), jnp.int32),
            pltpu.VMEM((WINDOW, EMBED), jnp.float32),
            pltpu.VMEM((ROWS, EMBED), jnp.float32),
            pltpu.VMEM((ROWS, EMBED), jnp.float32),
            pltpu.VMEM((ROWS, EMBED), jnp.float32),
            pltpu.VMEM((ROWS, EMBED), jnp.float32),
            pltpu.SemaphoreType.DMA,
            pltpu.SemaphoreType.DMA,
            pltpu.SemaphoreType.DMA,
            pltpu.SemaphoreType.DMA,
        ],
    )(xf, table, pe)
    return out
